# baseline (speedup 1.0000x reference)
import functools

import jax
import jax.numpy as jnp
from jax import lax
from jax.experimental import pallas as pl
from jax.experimental.pallas import tpu as pltpu
from jax.experimental.pallas import tpu_sc as plsc

N = 131072
C = 190
IGNORE = -1

NC = 2
NS = 16
L = 16
NW = NC * NS
CH = 128
NCH = 20
BPW = CH * NCH
N_SC = NW * BPW

BR = 4096
NB_ALL = N // BR
OFF_B = N_SC // BR
NB_TC = NB_ALL - OFF_B


def _sc_body(logits_hbm, tgt_hbm, sums_hbm, cnts_hbm, tgt_v, buf0, buf1,
             acc_v, cnt_v, sem0, sem1):
    c = lax.axis_index("c")
    s = lax.axis_index("s")
    wid = s * NC + c
    base = wid * BPW

    pltpu.sync_copy(tgt_hbm.at[pl.ds(base, BPW)], tgt_v)

    lanes = lax.iota(jnp.int32, L)
    bufs = (buf0, buf1)
    sems = (sem0, sem1)

    def start(k, buf, sem):
        pltpu.make_async_copy(
            logits_hbm.at[pl.ds(base + k * CH, CH), :], buf, sem).start()

    def drain(buf, sem):
        pltpu.make_async_copy(
            logits_hbm.at[pl.ds(base, CH), :], buf, sem).wait()

    start(0, bufs[0], sems[0])

    zero = jnp.zeros((L,), jnp.float32)
    acc, cnt = zero, zero
    for k in range(NCH):
        b = k & 1
        if k + 1 < NCH:
            start(k + 1, bufs[(k + 1) & 1], sems[(k + 1) & 1])
        drain(bufs[b], sems[b])

        def group(g, carry):
            a, n = carry
            t = tgt_v[pl.ds(k * CH + g * L, L)]
            keep = t != IGNORE
            safe = jnp.where(keep, t, 0)
            rows = g * L + lanes
            x = plsc.load_gather(bufs[b], [rows, safe])
            d = 1.0 - x
            a = a + jnp.where(keep, d * d, 0.0)
            n = n + jnp.where(keep, 1.0, 0.0)
            return a, n

        acc, cnt = lax.fori_loop(0, CH // L, group, (acc, cnt), unroll=4)

    acc_v[...] = acc
    cnt_v[...] = cnt
    pltpu.sync_copy(acc_v, sums_hbm.at[wid])
    pltpu.sync_copy(cnt_v, cnts_hbm.at[wid])


def _tc_body(tgt_ref, logits_ref, out_ref):
    x = logits_ref[...]
    t = tgt_ref[0, 0, :]
    t2 = t.reshape(BR, 1)
    cols = lax.broadcasted_iota(jnp.int32, (BR, C), 1)
    mask = cols == t2
    d = 1.0 - x
    ssum = jnp.sum(jnp.where(mask, d * d, 0.0))
    cnt = jnp.sum((tgt_ref[...] != IGNORE).astype(jnp.float32))
    lane = lax.broadcasted_iota(jnp.int32, (1, 1, 128), 2)
    out_ref[...] = jnp.where(lane == 0, ssum, jnp.where(lane == 1, cnt, 0.0))


@jax.jit
def _ppd_loss(logits, tgt):
    mesh = plsc.VectorSubcoreMesh(core_axis_name="c", subcore_axis_name="s")
    sc_sums, sc_cnts = pl.kernel(
        _sc_body,
        out_type=[
            jax.ShapeDtypeStruct((NW, L), jnp.float32),
            jax.ShapeDtypeStruct((NW, L), jnp.float32),
        ],
        mesh=mesh,
        compiler_params=pltpu.CompilerParams(needs_layout_passes=False),
        scratch_types=[
            pltpu.VMEM((BPW,), jnp.int32),
            pltpu.VMEM((CH, C), jnp.float32),
            pltpu.VMEM((CH, C), jnp.float32),
            pltpu.VMEM((L,), jnp.float32),
            pltpu.VMEM((L,), jnp.float32),
            pltpu.SemaphoreType.DMA,
            pltpu.SemaphoreType.DMA,
        ],
    )(logits, tgt)

    tgt3 = tgt.reshape(NB_ALL, 1, BR)
    tc_out = pl.pallas_call(
        _tc_body,
        grid=(NB_TC,),
        in_specs=[
            pl.BlockSpec((1, 1, BR), lambda b: (b + OFF_B, 0, 0)),
            pl.BlockSpec((BR, C), lambda b: (b + OFF_B, 0)),
        ],
        out_specs=pl.BlockSpec((1, 1, 128), lambda b: (b, 0, 0)),
        out_shape=jax.ShapeDtypeStruct((NB_TC, 1, 128), jnp.float32),
        compiler_params=pltpu.CompilerParams(
            dimension_semantics=("arbitrary",),
        ),
    )(tgt3, logits)

    total = jnp.sum(sc_sums) + jnp.sum(tc_out[:, 0, 0])
    count = jnp.sum(sc_cnts) + jnp.sum(tc_out[:, 0, 1])
    return total / count


def kernel(contrast_logits, contrast_target):
    return _ppd_loss(contrast_logits, contrast_target)


# speedup vs baseline: 1.0035x; 1.0035x over previous
import functools

import jax
import jax.numpy as jnp
from jax import lax
from jax.experimental import pallas as pl
from jax.experimental.pallas import tpu as pltpu
from jax.experimental.pallas import tpu_sc as plsc

N = 131072
C = 190
IGNORE = -1

NC = 2
NS = 16
L = 16
NW = NC * NS
CH = 128
NCH = 15
BPW = CH * NCH
N_SC = NW * BPW

BR = 4096
NB_ALL = N // BR
OFF_B = N_SC // BR
NB_TC = NB_ALL - OFF_B


def _sc_body(logits_hbm, tgt_hbm, sums_hbm, cnts_hbm, tgt_v, buf0, buf1,
             acc_v, cnt_v, sem0, sem1):
    c = lax.axis_index("c")
    s = lax.axis_index("s")
    wid = s * NC + c
    base = wid * BPW

    pltpu.sync_copy(tgt_hbm.at[pl.ds(base, BPW)], tgt_v)

    lanes = lax.iota(jnp.int32, L)
    bufs = (buf0, buf1)
    sems = (sem0, sem1)

    def start(k, buf, sem):
        pltpu.make_async_copy(
            logits_hbm.at[pl.ds(base + k * CH, CH), :], buf, sem).start()

    def drain(buf, sem):
        pltpu.make_async_copy(
            logits_hbm.at[pl.ds(base, CH), :], buf, sem).wait()

    start(0, bufs[0], sems[0])

    zero = jnp.zeros((L,), jnp.float32)
    acc, cnt = zero, zero
    for k in range(NCH):
        b = k & 1
        if k + 1 < NCH:
            start(k + 1, bufs[(k + 1) & 1], sems[(k + 1) & 1])
        drain(bufs[b], sems[b])

        def group(g, carry):
            a, n = carry
            t = tgt_v[pl.ds(k * CH + g * L, L)]
            keep = t != IGNORE
            safe = jnp.where(keep, t, 0)
            rows = g * L + lanes
            x = plsc.load_gather(bufs[b], [rows, safe])
            d = 1.0 - x
            a = a + jnp.where(keep, d * d, 0.0)
            n = n + jnp.where(keep, 1.0, 0.0)
            return a, n

        acc, cnt = lax.fori_loop(0, CH // L, group, (acc, cnt), unroll=4)

    acc_v[...] = acc
    cnt_v[...] = cnt
    pltpu.sync_copy(acc_v, sums_hbm.at[wid])
    pltpu.sync_copy(cnt_v, cnts_hbm.at[wid])


def _tc_body(tgt_ref, logits_ref, out_ref):
    x = logits_ref[...]
    t = tgt_ref[0, 0, :]
    t2 = t.reshape(BR, 1)
    cols = lax.broadcasted_iota(jnp.int32, (BR, C), 1)
    mask = cols == t2
    d = 1.0 - x
    ssum = jnp.sum(jnp.where(mask, d * d, 0.0))
    cnt = jnp.sum((tgt_ref[...] != IGNORE).astype(jnp.float32))
    lane = lax.broadcasted_iota(jnp.int32, (1, 1, 128), 2)
    out_ref[...] = jnp.where(lane == 0, ssum, jnp.where(lane == 1, cnt, 0.0))


@jax.jit
def _ppd_loss(logits, tgt):
    tgt3 = tgt.reshape(NB_ALL, 1, BR)
    tc_out = pl.pallas_call(
        _tc_body,
        grid=(NB_TC,),
        in_specs=[
            pl.BlockSpec((1, 1, BR), lambda b: (b + OFF_B, 0, 0)),
            pl.BlockSpec((BR, C), lambda b: (b + OFF_B, 0)),
        ],
        out_specs=pl.BlockSpec((1, 1, 128), lambda b: (b, 0, 0)),
        out_shape=jax.ShapeDtypeStruct((NB_TC, 1, 128), jnp.float32),
        compiler_params=pltpu.CompilerParams(
            dimension_semantics=("arbitrary",),
        ),
    )(tgt3, logits)

    mesh = plsc.VectorSubcoreMesh(core_axis_name="c", subcore_axis_name="s")
    sc_sums, sc_cnts = pl.kernel(
        _sc_body,
        out_type=[
            jax.ShapeDtypeStruct((NW, L), jnp.float32),
            jax.ShapeDtypeStruct((NW, L), jnp.float32),
        ],
        mesh=mesh,
        compiler_params=pltpu.CompilerParams(needs_layout_passes=False),
        scratch_types=[
            pltpu.VMEM((BPW,), jnp.int32),
            pltpu.VMEM((CH, C), jnp.float32),
            pltpu.VMEM((CH, C), jnp.float32),
            pltpu.VMEM((L,), jnp.float32),
            pltpu.VMEM((L,), jnp.float32),
            pltpu.SemaphoreType.DMA,
            pltpu.SemaphoreType.DMA,
        ],
    )(logits, tgt)

    total = jnp.sum(sc_sums) + jnp.sum(tc_out[:, 0, 0])
    count = jnp.sum(sc_cnts) + jnp.sum(tc_out[:, 0, 1])
    return total / count


def kernel(contrast_logits, contrast_target):
    return _ppd_loss(contrast_logits, contrast_target)


# speedup vs baseline: 1.0170x; 1.0135x over previous
import functools

import jax
import jax.numpy as jnp
from jax import lax
from jax.experimental import pallas as pl
from jax.experimental.pallas import tpu as pltpu
from jax.experimental.pallas import tpu_sc as plsc

N = 131072
C = 190
IGNORE = -1

NC = 2
NS = 16
L = 16
NW = NC * NS
CH = 128
NCH = 16
BPW = CH * NCH
N_SC = NW * BPW

BR = 8192
NB_ALL = N // BR
OFF_B = N_SC // BR
NB_TC = NB_ALL - OFF_B


def _sc_body(logits_hbm, tgt_hbm, sums_hbm, cnts_hbm, tgt_v, buf0, buf1,
             acc_v, cnt_v, sem0, sem1):
    c = lax.axis_index("c")
    s = lax.axis_index("s")
    wid = s * NC + c
    base = wid * BPW

    pltpu.sync_copy(tgt_hbm.at[pl.ds(base, BPW)], tgt_v)

    lanes = lax.iota(jnp.int32, L)
    bufs = (buf0, buf1)
    sems = (sem0, sem1)

    def start(k, buf, sem):
        pltpu.make_async_copy(
            logits_hbm.at[pl.ds(base + k * CH, CH), :], buf, sem).start()

    def drain(buf, sem):
        pltpu.make_async_copy(
            logits_hbm.at[pl.ds(base, CH), :], buf, sem).wait()

    start(0, bufs[0], sems[0])

    zero = jnp.zeros((L,), jnp.float32)
    acc, cnt = zero, zero
    for k in range(NCH):
        b = k & 1
        if k + 1 < NCH:
            start(k + 1, bufs[(k + 1) & 1], sems[(k + 1) & 1])
        drain(bufs[b], sems[b])

        def group(g, carry):
            a, n = carry
            t = tgt_v[pl.ds(k * CH + g * L, L)]
            keep = t != IGNORE
            safe = jnp.where(keep, t, 0)
            rows = g * L + lanes
            x = plsc.load_gather(bufs[b], [rows, safe])
            d = 1.0 - x
            a = a + jnp.where(keep, d * d, 0.0)
            n = n + jnp.where(keep, 1.0, 0.0)
            return a, n

        acc, cnt = lax.fori_loop(0, CH // L, group, (acc, cnt), unroll=4)

    acc_v[...] = acc
    cnt_v[...] = cnt
    pltpu.sync_copy(acc_v, sums_hbm.at[wid])
    pltpu.sync_copy(cnt_v, cnts_hbm.at[wid])


def _tc_body(tgt_ref, logits_ref, out_ref):
    x = logits_ref[...]
    t = tgt_ref[0, 0, :]
    t2 = t.reshape(BR, 1)
    cols = lax.broadcasted_iota(jnp.int32, (BR, C), 1)
    mask = cols == t2
    d = 1.0 - x
    ssum = jnp.sum(jnp.where(mask, d * d, 0.0))
    cnt = jnp.sum((tgt_ref[...] != IGNORE).astype(jnp.float32))
    lane = lax.broadcasted_iota(jnp.int32, (1, 1, 128), 2)
    out_ref[...] = jnp.where(lane == 0, ssum, jnp.where(lane == 1, cnt, 0.0))


@jax.jit
def _ppd_loss(logits, tgt):
    tgt3 = tgt.reshape(NB_ALL, 1, BR)
    tc_out = pl.pallas_call(
        _tc_body,
        grid=(NB_TC,),
        in_specs=[
            pl.BlockSpec((1, 1, BR), lambda b: (b + OFF_B, 0, 0)),
            pl.BlockSpec((BR, C), lambda b: (b + OFF_B, 0)),
        ],
        out_specs=pl.BlockSpec((1, 1, 128), lambda b: (b, 0, 0)),
        out_shape=jax.ShapeDtypeStruct((NB_TC, 1, 128), jnp.float32),
        compiler_params=pltpu.CompilerParams(
            dimension_semantics=("arbitrary",),
        ),
    )(tgt3, logits)

    mesh = plsc.VectorSubcoreMesh(core_axis_name="c", subcore_axis_name="s")
    sc_sums, sc_cnts = pl.kernel(
        _sc_body,
        out_type=[
            jax.ShapeDtypeStruct((NW, L), jnp.float32),
            jax.ShapeDtypeStruct((NW, L), jnp.float32),
        ],
        mesh=mesh,
        compiler_params=pltpu.CompilerParams(needs_layout_passes=False),
        scratch_types=[
            pltpu.VMEM((BPW,), jnp.int32),
            pltpu.VMEM((CH, C), jnp.float32),
            pltpu.VMEM((CH, C), jnp.float32),
            pltpu.VMEM((L,), jnp.float32),
            pltpu.VMEM((L,), jnp.float32),
            pltpu.SemaphoreType.DMA,
            pltpu.SemaphoreType.DMA,
        ],
    )(logits, tgt)

    total = jnp.sum(sc_sums) + jnp.sum(tc_out[:, 0, 0])
    count = jnp.sum(sc_cnts) + jnp.sum(tc_out[:, 0, 1])
    return total / count


def kernel(contrast_logits, contrast_target):
    return _ppd_loss(contrast_logits, contrast_target)
